# Initial kernel scaffold; baseline (speedup 1.0000x reference)
import jax
import jax.numpy as jnp
from jax.experimental import pallas as pl


def kernel(ij, r, m, i, j, v, mu, w, b):
    raise NotImplementedError("write your pallas kernel here")


# speedup vs baseline: 3.9675x; 3.9675x over previous
import functools

import jax
import jax.numpy as jnp
from jax import lax
from jax.experimental import pallas as pl
from jax.experimental.pallas import tpu as pltpu
from jax.experimental.pallas import tpu_sc as plsc

_N = 10000
_D = 128
_M = 10000
_NNZ = 320000
_B = 16384

_NC = 2
_NS = 16
_NW = _NC * _NS
_L = 16
_Q = _D // _L

_ET = _NNZ // _NW
_C1 = 80
_NCH1 = _ET // _C1
_ROWS_T = 640
_MP = _NS * _ROWS_T
_RCH = 128
_NRCH = _ROWS_T // _RCH

_PT = _B // _NW
_C2 = 64
_NCH2 = _PT // _C2


def _mesh():
    return plsc.VectorSubcoreMesh(
        core_axis_name="c", subcore_axis_name="s",
        num_cores=_NC, num_subcores=_NS)


@functools.partial(
    pl.kernel,
    out_type=jax.ShapeDtypeStruct((_NC * _MP, _D), jnp.float32),
    mesh=_mesh(),
    scratch_types=[
        pltpu.VMEM_SHARED((_MP, _D), jnp.float32),
        pltpu.VMEM((_C1,), jnp.int32),
        pltpu.VMEM((_C1,), jnp.int32),
        pltpu.VMEM((_C1,), jnp.float32),
        pltpu.VMEM((_C1, _D), jnp.float32),
        pltpu.VMEM((_RCH, _D), jnp.float32),
        pltpu.SemaphoreType.DMA,
    ],
)
def _spmm(rows_hbm, cols_hbm, rv_hbm, v_hbm, aggp_hbm,
          agg_sh, rowbuf, colbuf, rvm, gbuf, tbuf, sem):
    cc = lax.axis_index("c")
    ss = lax.axis_index("s")
    wid = cc * _NS + ss
    ebase = wid * _ET

    @pl.loop(0, _RCH)
    def _zrow(rr):
        for q in range(_Q):
            tbuf[rr, pl.ds(q * _L, _L)] = jnp.zeros((_L,), jnp.float32)

    @pl.loop(0, _NRCH)
    def _zcopy(k):
        r0 = ss * _ROWS_T + k * _RCH
        pltpu.sync_copy(tbuf, agg_sh.at[pl.ds(r0, _RCH)])

    plsc.subcore_barrier()

    @pl.loop(0, _NCH1)
    def _chunk(g):
        off = pl.multiple_of(ebase + g * _C1, 8)
        pltpu.sync_copy(rows_hbm.at[pl.ds(off, _C1)], rowbuf)
        pltpu.sync_copy(cols_hbm.at[pl.ds(off, _C1)], colbuf)
        pltpu.sync_copy(rv_hbm.at[pl.ds(off, _C1)], rvm)
        pltpu.async_copy(v_hbm.at[colbuf], gbuf, sem).wait()

        @pl.loop(0, _C1 // _L)
        def _scale(eb):
            rv16 = rvm[pl.ds(eb * _L, _L)]
            for k in range(_L):
                rvv = jnp.full((_L,), rv16[k], jnp.float32)
                e = eb * _L + k
                for q in range(_Q):
                    sl = pl.ds(q * _L, _L)
                    gbuf[e, sl] = gbuf[e, sl] * rvv

        pltpu.sync_copy(gbuf, agg_sh.at[rowbuf], add=True)

    plsc.subcore_barrier()

    @pl.loop(0, _NRCH)
    def _ocopy(k):
        r0 = ss * _ROWS_T + k * _RCH
        pltpu.sync_copy(agg_sh.at[pl.ds(r0, _RCH)], tbuf)
        pltpu.sync_copy(tbuf, aggp_hbm.at[pl.ds(cc * _MP + r0, _RCH)])


@functools.partial(
    pl.kernel,
    out_type=(jax.ShapeDtypeStruct((_NW * _L,), jnp.float32),
              jax.ShapeDtypeStruct((_B,), jnp.float32)),
    mesh=_mesh(),
    scratch_types=[
        pltpu.VMEM((_C2,), jnp.int32),
        pltpu.VMEM((_C2,), jnp.int32),
        pltpu.VMEM((_C2,), jnp.int32),
        pltpu.VMEM((_C2, _D), jnp.float32),
        pltpu.VMEM((_C2, _D), jnp.float32),
        pltpu.VMEM((_C2, _D), jnp.float32),
        pltpu.VMEM((1, _D), jnp.float32),
        pltpu.VMEM((_PT,), jnp.float32),
        pltpu.VMEM((_L,), jnp.float32),
        pltpu.SemaphoreType.DMA,
        pltpu.SemaphoreType.DMA,
        pltpu.SemaphoreType.DMA,
        pltpu.SemaphoreType.DMA,
    ],
)
def _pairs(aggp_hbm, w_hbm, b_hbm, mu_hbm, i_hbm, j_hbm,
           pd_hbm, bj_hbm,
           ibuf0, ibuf1, jbuf, a0, a1, wbuf, mubuf, bjbuf, pdbuf,
           sem0, sem1, sem2, sem3):
    cc = lax.axis_index("c")
    ss = lax.axis_index("s")
    wid = cc * _NS + ss

    pltpu.sync_copy(mu_hbm, mubuf)

    acc = tuple(jnp.zeros((_L,), jnp.float32) for _ in range(_Q))
    for g in range(_NCH2):
        off = pl.multiple_of(wid * _PT + g * _C2, 8)
        pltpu.sync_copy(i_hbm.at[pl.ds(off, _C2)], ibuf0)
        pltpu.sync_copy(j_hbm.at[pl.ds(off, _C2)], jbuf)
        for t in range(_C2 // _L):
            sl = pl.ds(t * _L, _L)
            ibuf1[sl] = ibuf0[sl] + _MP
        d0 = pltpu.async_copy(aggp_hbm.at[ibuf0], a0, sem0)
        d1 = pltpu.async_copy(aggp_hbm.at[ibuf1], a1, sem1)
        d2 = pltpu.async_copy(w_hbm.at[jbuf], wbuf, sem2)
        d3 = pltpu.async_copy(b_hbm.at[jbuf],
                              bjbuf.at[pl.ds(g * _C2, _C2)], sem3)
        d0.wait()
        d1.wait()
        d2.wait()
        d3.wait()

        def _inner(e, acc):
            new = []
            for q in range(_Q):
                sl = pl.ds(q * _L, _L)
                x = a0[e, sl] + a1[e, sl] + mubuf[0, sl]
                h = 1.0 / (1.0 + jnp.exp(-x))
                new.append(acc[q] + h * wbuf[e, sl])
            return tuple(new)

        acc = lax.fori_loop(0, _C2, _inner, acc)

    tot = acc[0]
    for q in range(1, _Q):
        tot = tot + acc[q]
    pdbuf[...] = tot
    pltpu.sync_copy(pdbuf, pd_hbm.at[pl.ds(wid * _L, _L)])
    pltpu.sync_copy(bjbuf, bj_hbm.at[pl.ds(wid * _PT, _PT)])


def kernel(ij, r, m, i, j, v, mu, w, b):
    del m
    ij = ij.astype(jnp.int32)
    rows = ij[0]
    cols = ij[1]
    aggp = _spmm(rows, cols, r.astype(jnp.float32), v)
    pd, bj = _pairs(aggp, w, b, mu, i.astype(jnp.int32), j.astype(jnp.int32))
    return jnp.sum(pd) + bj


# speedup vs baseline: 8.3835x; 2.1130x over previous
import functools

import jax
import jax.numpy as jnp
from jax import lax
from jax.experimental import pallas as pl
from jax.experimental.pallas import tpu as pltpu
from jax.experimental.pallas import tpu_sc as plsc

_N = 10000
_D = 128
_M = 10000
_NNZ = 320000
_B = 16384

_NC = 2
_NS = 16
_NW = _NC * _NS
_L = 16
_Q = _D // _L

_ET = _NNZ // _NW
_C1 = 80
_NCH1 = _ET // _C1
_NPAIR = (_NCH1 - 1) // 2
_ROWS_T = 640
_MP = _NS * _ROWS_T
_RCH = _C1
_NRCH = _ROWS_T // _RCH

_PT = _B // _NW
_C2 = 64
_NCH2 = _PT // _C2


def _mesh():
    return plsc.VectorSubcoreMesh(
        core_axis_name="c", subcore_axis_name="s",
        num_cores=_NC, num_subcores=_NS)


@functools.partial(
    pl.kernel,
    out_type=jax.ShapeDtypeStruct((_NC * _MP, _D), jnp.float32),
    mesh=_mesh(),
    scratch_types=[
        pltpu.VMEM_SHARED((_MP, _D), jnp.float32),
        pltpu.VMEM((_ET,), jnp.int32),
        pltpu.VMEM((_C1,), jnp.int32),
        pltpu.VMEM((_C1,), jnp.int32),
        pltpu.VMEM((_C1,), jnp.float32),
        pltpu.VMEM((_C1,), jnp.float32),
        pltpu.VMEM((_C1, _D), jnp.float32),
        pltpu.VMEM((_C1, _D), jnp.float32),
        pltpu.SemaphoreType.DMA,
        pltpu.SemaphoreType.DMA,
        pltpu.SemaphoreType.DMA,
        pltpu.SemaphoreType.DMA,
        pltpu.SemaphoreType.DMA,
        pltpu.SemaphoreType.DMA,
        pltpu.SemaphoreType.DMA,
    ],
)
def _spmm(rows_hbm, cols_hbm, rv_hbm, v_hbm, aggp_hbm,
          agg_sh, cols_l, rowsA, rowsB, rA, rB, gbA, gbB,
          semi, semIA, semIB, semA, semB, semSA, semSB):
    cc = lax.axis_index("c")
    ss = lax.axis_index("s")
    wid = cc * _NS + ss
    ebase = wid * _ET

    dc = pltpu.async_copy(cols_hbm.at[pl.ds(ebase, _ET)], cols_l, semi)

    @pl.loop(0, _C1)
    def _zrow(rr):
        for q in range(_Q):
            gbA[rr, pl.ds(q * _L, _L)] = jnp.zeros((_L,), jnp.float32)

    @pl.loop(0, _NRCH)
    def _zcopy(k):
        r0 = ss * _ROWS_T + k * _RCH
        pltpu.sync_copy(gbA, agg_sh.at[pl.ds(r0, _RCH)])

    dc.wait()
    plsc.subcore_barrier()

    def _idx_start(g, rowsX, rX, semIX):
        off = pl.multiple_of(ebase + g * _C1, 8)
        pltpu.async_copy(rows_hbm.at[pl.ds(off, _C1)], rowsX, semIX)
        pltpu.async_copy(rv_hbm.at[pl.ds(off, _C1)], rX, semIX)

    def _idx_wait(g, rowsX, rX, semIX):
        off = pl.multiple_of(ebase + g * _C1, 8)
        pltpu.make_async_copy(rows_hbm.at[pl.ds(off, _C1)], rowsX, semIX).wait()
        pltpu.make_async_copy(rv_hbm.at[pl.ds(off, _C1)], rX, semIX).wait()

    def _gather_start(g, gb, sem):
        off = pl.multiple_of(g * _C1, 8)
        pltpu.async_copy(v_hbm.at[cols_l.at[pl.ds(off, _C1)]], gb, sem)

    def _gather_wait(g, gb, sem):
        off = pl.multiple_of(g * _C1, 8)
        pltpu.make_async_copy(v_hbm.at[cols_l.at[pl.ds(off, _C1)]], gb, sem).wait()

    def _scale(gb, rX):
        @pl.loop(0, _C1 // _L)
        def _s(eb):
            rv16 = rX[pl.ds(eb * _L, _L)]
            for k in range(_L):
                rvv = jnp.full((_L,), rv16[k], jnp.float32)
                e = eb * _L + k
                for q in range(_Q):
                    sl = pl.ds(q * _L, _L)
                    gb[e, sl] = gb[e, sl] * rvv

    def _scatter_start(gb, rowsX, semSX):
        pltpu.async_copy(gb, agg_sh.at[rowsX], semSX, add=True)

    def _scatter_wait(gb, rowsX, semSX):
        pltpu.make_async_copy(gb, agg_sh.at[rowsX], semSX).wait()

    _idx_start(0, rowsA, rA, semIA)
    _gather_start(0, gbA, semA)
    _idx_start(1, rowsB, rB, semIB)
    _gather_start(1, gbB, semB)

    @pl.loop(0, _NPAIR)
    def _pair(p):
        g0 = 2 * p
        g1 = g0 + 1
        _gather_wait(g0, gbA, semA)
        _idx_wait(g0, rowsA, rA, semIA)
        _scale(gbA, rA)
        _scatter_start(gbA, rowsA, semSA)
        _gather_wait(g1, gbB, semB)
        _idx_wait(g1, rowsB, rB, semIB)
        _scale(gbB, rB)
        _scatter_start(gbB, rowsB, semSB)
        _scatter_wait(gbA, rowsA, semSA)
        _idx_start(g0 + 2, rowsA, rA, semIA)
        _gather_start(g0 + 2, gbA, semA)
        _scatter_wait(gbB, rowsB, semSB)

        @pl.when(g1 + 2 < _NCH1)
        def _():
            _idx_start(g1 + 2, rowsB, rB, semIB)
            _gather_start(g1 + 2, gbB, semB)

    glast = _NCH1 - 1
    _gather_wait(glast, gbA, semA)
    _idx_wait(glast, rowsA, rA, semIA)
    _scale(gbA, rA)
    _scatter_start(gbA, rowsA, semSA)
    _scatter_wait(gbA, rowsA, semSA)

    plsc.subcore_barrier()

    @pl.loop(0, _NRCH)
    def _ocopy(k):
        r0 = ss * _ROWS_T + k * _RCH
        pltpu.sync_copy(agg_sh.at[pl.ds(r0, _RCH)], gbA)
        pltpu.sync_copy(gbA, aggp_hbm.at[pl.ds(cc * _MP + r0, _RCH)])


@functools.partial(
    pl.kernel,
    out_type=(jax.ShapeDtypeStruct((_NW * _L,), jnp.float32),
              jax.ShapeDtypeStruct((_B,), jnp.float32)),
    mesh=_mesh(),
    scratch_types=[
        pltpu.VMEM((_C2,), jnp.int32),
        pltpu.VMEM((_C2,), jnp.int32),
        pltpu.VMEM((_C2,), jnp.int32),
        pltpu.VMEM((_C2, _D), jnp.float32),
        pltpu.VMEM((_C2, _D), jnp.float32),
        pltpu.VMEM((_C2, _D), jnp.float32),
        pltpu.VMEM((1, _D), jnp.float32),
        pltpu.VMEM((_PT,), jnp.float32),
        pltpu.VMEM((_L,), jnp.float32),
        pltpu.SemaphoreType.DMA,
        pltpu.SemaphoreType.DMA,
        pltpu.SemaphoreType.DMA,
        pltpu.SemaphoreType.DMA,
    ],
)
def _pairs(aggp_hbm, w_hbm, b_hbm, mu_hbm, i_hbm, j_hbm,
           pd_hbm, bj_hbm,
           ibuf0, ibuf1, jbuf, a0, a1, wbuf, mubuf, bjbuf, pdbuf,
           sem0, sem1, sem2, sem3):
    cc = lax.axis_index("c")
    ss = lax.axis_index("s")
    wid = cc * _NS + ss

    pltpu.sync_copy(mu_hbm, mubuf)

    acc = tuple(jnp.zeros((_L,), jnp.float32) for _ in range(_Q))
    for g in range(_NCH2):
        off = pl.multiple_of(wid * _PT + g * _C2, 8)
        pltpu.sync_copy(i_hbm.at[pl.ds(off, _C2)], ibuf0)
        pltpu.sync_copy(j_hbm.at[pl.ds(off, _C2)], jbuf)
        for t in range(_C2 // _L):
            sl = pl.ds(t * _L, _L)
            ibuf1[sl] = ibuf0[sl] + _MP
        d0 = pltpu.async_copy(aggp_hbm.at[ibuf0], a0, sem0)
        d1 = pltpu.async_copy(aggp_hbm.at[ibuf1], a1, sem1)
        d2 = pltpu.async_copy(w_hbm.at[jbuf], wbuf, sem2)
        d3 = pltpu.async_copy(b_hbm.at[jbuf],
                              bjbuf.at[pl.ds(g * _C2, _C2)], sem3)
        d0.wait()
        d1.wait()
        d2.wait()
        d3.wait()

        def _inner(e, acc):
            new = []
            for q in range(_Q):
                sl = pl.ds(q * _L, _L)
                x = a0[e, sl] + a1[e, sl] + mubuf[0, sl]
                h = 1.0 / (1.0 + jnp.exp(-x))
                new.append(acc[q] + h * wbuf[e, sl])
            return tuple(new)

        acc = lax.fori_loop(0, _C2, _inner, acc)

    tot = acc[0]
    for q in range(1, _Q):
        tot = tot + acc[q]
    pdbuf[...] = tot
    pltpu.sync_copy(pdbuf, pd_hbm.at[pl.ds(wid * _L, _L)])
    pltpu.sync_copy(bjbuf, bj_hbm.at[pl.ds(wid * _PT, _PT)])


def kernel(ij, r, m, i, j, v, mu, w, b):
    del m
    ij = ij.astype(jnp.int32)
    rows = ij[0]
    cols = ij[1]
    aggp = _spmm(rows, cols, r.astype(jnp.float32), v)
    pd, bj = _pairs(aggp, w, b, mu, i.astype(jnp.int32), j.astype(jnp.int32))
    return jnp.sum(pd) + bj


# speedup vs baseline: 9.0581x; 1.0805x over previous
import functools

import jax
import jax.numpy as jnp
from jax import lax
from jax.experimental import pallas as pl
from jax.experimental.pallas import tpu as pltpu
from jax.experimental.pallas import tpu_sc as plsc

_N = 10000
_D = 128
_M = 10000
_NNZ = 320000
_B = 16384

_NC = 2
_NS = 16
_NW = _NC * _NS
_L = 16
_Q = _D // _L

_ET = _NNZ // _NW
_C1 = 80
_NCH1 = _ET // _C1
_NPAIR = (_NCH1 - 1) // 2
_ROWS_T = 640
_MP = _NS * _ROWS_T
_RCH = _C1
_NRCH = _ROWS_T // _RCH

_PT = _B // _NW
_C2 = 64
_NCH2 = _PT // _C2


def _mesh():
    return plsc.VectorSubcoreMesh(
        core_axis_name="c", subcore_axis_name="s",
        num_cores=_NC, num_subcores=_NS)


@functools.partial(
    pl.kernel,
    out_type=jax.ShapeDtypeStruct((_NC * _MP, _D), jnp.float32),
    mesh=_mesh(),
    scratch_types=[
        pltpu.VMEM_SHARED((_MP, _D), jnp.float32),
        pltpu.VMEM((_ET,), jnp.int32),
        pltpu.VMEM((_C1,), jnp.int32),
        pltpu.VMEM((_C1,), jnp.int32),
        pltpu.VMEM((_C1,), jnp.float32),
        pltpu.VMEM((_C1,), jnp.float32),
        pltpu.VMEM((_C1, _D), jnp.float32),
        pltpu.VMEM((_C1, _D), jnp.float32),
        pltpu.SemaphoreType.DMA,
        pltpu.SemaphoreType.DMA,
        pltpu.SemaphoreType.DMA,
        pltpu.SemaphoreType.DMA,
        pltpu.SemaphoreType.DMA,
        pltpu.SemaphoreType.DMA,
        pltpu.SemaphoreType.DMA,
    ],
)
def _spmm(rows_hbm, cols_hbm, rv_hbm, v_hbm, aggp_hbm,
          agg_sh, cols_l, rowsA, rowsB, rA, rB, gbA, gbB,
          semi, semIA, semIB, semA, semB, semSA, semSB):
    cc = lax.axis_index("c")
    ss = lax.axis_index("s")
    wid = cc * _NS + ss
    ebase = wid * _ET

    dc = pltpu.async_copy(cols_hbm.at[pl.ds(ebase, _ET)], cols_l, semi)

    @pl.loop(0, _C1)
    def _zrow(rr):
        for q in range(_Q):
            gbA[rr, pl.ds(q * _L, _L)] = jnp.zeros((_L,), jnp.float32)

    @pl.loop(0, _NRCH)
    def _zcopy(k):
        r0 = ss * _ROWS_T + k * _RCH
        pltpu.sync_copy(gbA, agg_sh.at[pl.ds(r0, _RCH)])

    dc.wait()
    plsc.subcore_barrier()

    def _idx_start(g, rowsX, rX, semIX):
        off = pl.multiple_of(ebase + g * _C1, 8)
        pltpu.async_copy(rows_hbm.at[pl.ds(off, _C1)], rowsX, semIX)
        pltpu.async_copy(rv_hbm.at[pl.ds(off, _C1)], rX, semIX)

    def _idx_wait(g, rowsX, rX, semIX):
        off = pl.multiple_of(ebase + g * _C1, 8)
        pltpu.make_async_copy(rows_hbm.at[pl.ds(off, _C1)], rowsX, semIX).wait()
        pltpu.make_async_copy(rv_hbm.at[pl.ds(off, _C1)], rX, semIX).wait()

    def _gather_start(g, gb, sem):
        off = pl.multiple_of(g * _C1, 8)
        pltpu.async_copy(v_hbm.at[cols_l.at[pl.ds(off, _C1)]], gb, sem)

    def _gather_wait(g, gb, sem):
        off = pl.multiple_of(g * _C1, 8)
        pltpu.make_async_copy(v_hbm.at[cols_l.at[pl.ds(off, _C1)]], gb, sem).wait()

    def _scale(gb, rX):
        @pl.loop(0, _C1 // _L)
        def _s(eb):
            rv16 = rX[pl.ds(eb * _L, _L)]
            for k in range(_L):
                rvv = jnp.full((_L,), rv16[k], jnp.float32)
                e = eb * _L + k
                for q in range(_Q):
                    sl = pl.ds(q * _L, _L)
                    gb[e, sl] = gb[e, sl] * rvv

    def _scatter_start(gb, rowsX, semSX):
        pltpu.async_copy(gb, agg_sh.at[rowsX], semSX, add=True)

    def _scatter_wait(gb, rowsX, semSX):
        pltpu.make_async_copy(gb, agg_sh.at[rowsX], semSX).wait()

    _idx_start(0, rowsA, rA, semIA)
    _gather_start(0, gbA, semA)
    _idx_start(1, rowsB, rB, semIB)
    _gather_start(1, gbB, semB)

    @pl.loop(0, _NPAIR)
    def _pair(p):
        g0 = 2 * p
        g1 = g0 + 1
        _gather_wait(g0, gbA, semA)
        _idx_wait(g0, rowsA, rA, semIA)
        _scale(gbA, rA)
        _scatter_start(gbA, rowsA, semSA)
        _gather_wait(g1, gbB, semB)
        _idx_wait(g1, rowsB, rB, semIB)
        _scale(gbB, rB)
        _scatter_start(gbB, rowsB, semSB)
        _scatter_wait(gbA, rowsA, semSA)
        _idx_start(g0 + 2, rowsA, rA, semIA)
        _gather_start(g0 + 2, gbA, semA)
        _scatter_wait(gbB, rowsB, semSB)

        @pl.when(g1 + 2 < _NCH1)
        def _():
            _idx_start(g1 + 2, rowsB, rB, semIB)
            _gather_start(g1 + 2, gbB, semB)

    glast = _NCH1 - 1
    _gather_wait(glast, gbA, semA)
    _idx_wait(glast, rowsA, rA, semIA)
    _scale(gbA, rA)
    _scatter_start(gbA, rowsA, semSA)
    _scatter_wait(gbA, rowsA, semSA)

    plsc.subcore_barrier()

    @pl.loop(0, _NRCH)
    def _ocopy(k):
        r0 = ss * _ROWS_T + k * _RCH
        pltpu.sync_copy(agg_sh.at[pl.ds(r0, _RCH)], gbA)
        pltpu.sync_copy(gbA, aggp_hbm.at[pl.ds(cc * _MP + r0, _RCH)])


@functools.partial(
    pl.kernel,
    out_type=(jax.ShapeDtypeStruct((_NW * _L,), jnp.float32),
              jax.ShapeDtypeStruct((_B,), jnp.float32)),
    mesh=_mesh(),
    scratch_types=[
        pltpu.VMEM((_C2,), jnp.int32),
        pltpu.VMEM((_C2,), jnp.int32),
        pltpu.VMEM((_C2,), jnp.int32),
        pltpu.VMEM((_C2,), jnp.int32),
        pltpu.VMEM((_C2,), jnp.int32),
        pltpu.VMEM((_C2,), jnp.int32),
        pltpu.VMEM((_C2, _D), jnp.float32),
        pltpu.VMEM((_C2, _D), jnp.float32),
        pltpu.VMEM((_C2, _D), jnp.float32),
        pltpu.VMEM((_C2, _D), jnp.float32),
        pltpu.VMEM((_C2, _D), jnp.float32),
        pltpu.VMEM((_C2, _D), jnp.float32),
        pltpu.VMEM((1, _D), jnp.float32),
        pltpu.VMEM((_PT,), jnp.float32),
        pltpu.VMEM((_L,), jnp.float32),
        pltpu.SemaphoreType.DMA,
        pltpu.SemaphoreType.DMA,
        pltpu.SemaphoreType.DMA,
        pltpu.SemaphoreType.DMA,
    ],
)
def _pairs(aggp_hbm, w_hbm, b_hbm, mu_hbm, i_hbm, j_hbm,
           pd_hbm, bj_hbm,
           iA, iB, i1A, i1B, jA, jB, a0A, a0B, a1A, a1B, wA, wB,
           mubuf, bjbuf, pdbuf, semIA, semIB, semGA, semGB):
    cc = lax.axis_index("c")
    ss = lax.axis_index("s")
    wid = cc * _NS + ss

    pltpu.sync_copy(mu_hbm, mubuf)

    def _idx_start(g, iX, jX, semIX):
        off = pl.multiple_of(wid * _PT + g * _C2, 8)
        pltpu.async_copy(i_hbm.at[pl.ds(off, _C2)], iX, semIX)
        pltpu.async_copy(j_hbm.at[pl.ds(off, _C2)], jX, semIX)

    def _idx_wait(g, iX, jX, semIX):
        off = pl.multiple_of(wid * _PT + g * _C2, 8)
        pltpu.make_async_copy(i_hbm.at[pl.ds(off, _C2)], iX, semIX).wait()
        pltpu.make_async_copy(j_hbm.at[pl.ds(off, _C2)], jX, semIX).wait()

    def _gath_start(g, iX, i1X, jX, a0X, a1X, wX, semGX):
        for t in range(_C2 // _L):
            sl = pl.ds(t * _L, _L)
            i1X[sl] = iX[sl] + _MP
        pltpu.async_copy(aggp_hbm.at[iX], a0X, semGX)
        pltpu.async_copy(aggp_hbm.at[i1X], a1X, semGX)
        pltpu.async_copy(w_hbm.at[jX], wX, semGX)
        pltpu.async_copy(b_hbm.at[jX],
                         bjbuf.at[pl.ds(g * _C2, _C2)], semGX)

    def _gath_wait(g, iX, i1X, jX, a0X, a1X, wX, semGX):
        pltpu.make_async_copy(aggp_hbm.at[iX], a0X, semGX).wait()
        pltpu.make_async_copy(aggp_hbm.at[i1X], a1X, semGX).wait()
        pltpu.make_async_copy(w_hbm.at[jX], wX, semGX).wait()
        pltpu.make_async_copy(b_hbm.at[jX],
                              bjbuf.at[pl.ds(g * _C2, _C2)], semGX).wait()

    def _compute(a0X, a1X, wX, acc):
        def _inner(e, acc):
            new = []
            for q in range(_Q):
                sl = pl.ds(q * _L, _L)
                x = a0X[e, sl] + a1X[e, sl] + mubuf[0, sl]
                h = 1.0 / (1.0 + jnp.exp(-x))
                new.append(acc[q] + h * wX[e, sl])
            return tuple(new)

        return lax.fori_loop(0, _C2, _inner, acc)

    _idx_start(0, iA, jA, semIA)
    _idx_wait(0, iA, jA, semIA)
    _gath_start(0, iA, i1A, jA, a0A, a1A, wA, semGA)
    _idx_start(1, iB, jB, semIB)
    _idx_wait(1, iB, jB, semIB)
    _gath_start(1, iB, i1B, jB, a0B, a1B, wB, semGB)

    acc0 = tuple(jnp.zeros((_L,), jnp.float32) for _ in range(_Q))

    @pl.loop(0, _NCH2 // 2, init_carry=acc0)
    def _pairloop(p, acc):
        g0 = 2 * p
        g1 = g0 + 1
        _gath_wait(g0, iA, i1A, jA, a0A, a1A, wA, semGA)

        @pl.when(g0 + 2 < _NCH2)
        def _():
            _idx_start(g0 + 2, iA, jA, semIA)

        acc = _compute(a0A, a1A, wA, acc)
        _gath_wait(g1, iB, i1B, jB, a0B, a1B, wB, semGB)

        @pl.when(g0 + 2 < _NCH2)
        def _():
            _idx_wait(g0 + 2, iA, jA, semIA)
            _gath_start(g0 + 2, iA, i1A, jA, a0A, a1A, wA, semGA)
            _idx_start(g1 + 2, iB, jB, semIB)

        acc = _compute(a0B, a1B, wB, acc)

        @pl.when(g1 + 2 < _NCH2)
        def _():
            _idx_wait(g1 + 2, iB, jB, semIB)
            _gath_start(g1 + 2, iB, i1B, jB, a0B, a1B, wB, semGB)

        return acc

    acc = _pairloop
    tot = acc[0]
    for q in range(1, _Q):
        tot = tot + acc[q]
    pdbuf[...] = tot
    pltpu.sync_copy(pdbuf, pd_hbm.at[pl.ds(wid * _L, _L)])
    pltpu.sync_copy(bjbuf, bj_hbm.at[pl.ds(wid * _PT, _PT)])


def kernel(ij, r, m, i, j, v, mu, w, b):
    del m
    ij = ij.astype(jnp.int32)
    rows = ij[0]
    cols = ij[1]
    aggp = _spmm(rows, cols, r.astype(jnp.float32), v)
    pd, bj = _pairs(aggp, w, b, mu, i.astype(jnp.int32), j.astype(jnp.int32))
    return jnp.sum(pd) + bj


# speedup vs baseline: 9.0695x; 1.0013x over previous
import functools

import jax
import jax.numpy as jnp
from jax import lax
from jax.experimental import pallas as pl
from jax.experimental.pallas import tpu as pltpu
from jax.experimental.pallas import tpu_sc as plsc

_N = 10000
_D = 128
_M = 10000
_NNZ = 320000
_B = 16384

_NC = 2
_NS = 16
_NW = _NC * _NS
_L = 16
_Q = _D // _L

_ET = _NNZ // _NW
_C1 = 80
_NCH1 = _ET // _C1
_NPAIR = (_NCH1 - 1) // 2
_ROWS_T = 640
_MP = _NS * _ROWS_T
_RCH = 128
_NRCH = _ROWS_T // _RCH

_PT = _B // _NW
_C2 = 64
_NCH2 = _PT // _C2


def _mesh():
    return plsc.VectorSubcoreMesh(
        core_axis_name="c", subcore_axis_name="s",
        num_cores=_NC, num_subcores=_NS)


_CC = 16
_NCHD = _ET // _CC
_RING = 8
_NGRP = 78


@functools.partial(
    pl.kernel,
    out_type=jax.ShapeDtypeStruct((_NC * _MP, _D), jnp.float32),
    mesh=_mesh(),
    scratch_types=[
        pltpu.VMEM_SHARED((_MP, _D), jnp.float32),
        pltpu.VMEM((_ET,), jnp.int32),
        pltpu.VMEM((_RING, _CC), jnp.int32),
        pltpu.VMEM((_RING, _CC), jnp.float32),
        pltpu.VMEM((_RING * _CC, _D), jnp.float32),
        pltpu.SemaphoreType.DMA,
        [pltpu.SemaphoreType.DMA] * _RING,
        [pltpu.SemaphoreType.DMA] * _RING,
    ],
)
def _spmm(rows_hbm, cols_hbm, rv_hbm, v_hbm, aggp_hbm,
          agg_sh, cols_l, rowsbuf, rvbuf, gb, semi, semIn, semOut):
    cc = lax.axis_index("c")
    ss = lax.axis_index("s")
    wid = cc * _NS + ss
    ebase = wid * _ET

    dc = pltpu.async_copy(cols_hbm.at[pl.ds(ebase, _ET)], cols_l, semi)

    @pl.loop(0, _RCH)
    def _zrow(rr):
        for q in range(_Q):
            gb[rr, pl.ds(q * _L, _L)] = jnp.zeros((_L,), jnp.float32)

    @pl.loop(0, _NRCH)
    def _zcopy(k):
        r0 = ss * _ROWS_T + k * _RCH
        pltpu.sync_copy(gb.at[pl.ds(0, _RCH)], agg_sh.at[pl.ds(r0, _RCH)])

    dc.wait()
    plsc.subcore_barrier()

    def _gs(g, b):
        off = pl.multiple_of(g * _CC, 8)
        hoff = pl.multiple_of(ebase + g * _CC, 8)
        pltpu.async_copy(rows_hbm.at[pl.ds(hoff, _CC)], rowsbuf.at[b], semIn[b])
        pltpu.async_copy(rv_hbm.at[pl.ds(hoff, _CC)], rvbuf.at[b], semIn[b])
        pltpu.async_copy(v_hbm.at[cols_l.at[pl.ds(off, _CC)]],
                         gb.at[pl.ds(b * _CC, _CC)], semIn[b])

    def _gw(g, b):
        off = pl.multiple_of(g * _CC, 8)
        hoff = pl.multiple_of(ebase + g * _CC, 8)
        pltpu.make_async_copy(rows_hbm.at[pl.ds(hoff, _CC)], rowsbuf.at[b],
                              semIn[b]).wait()
        pltpu.make_async_copy(rv_hbm.at[pl.ds(hoff, _CC)], rvbuf.at[b],
                              semIn[b]).wait()
        pltpu.make_async_copy(v_hbm.at[cols_l.at[pl.ds(off, _CC)]],
                              gb.at[pl.ds(b * _CC, _CC)], semIn[b]).wait()

    def _scale16(b):
        rv16 = rvbuf[b, pl.ds(0, _CC)]
        for k in range(_CC):
            rvv = jnp.full((_L,), rv16[k], jnp.float32)
            e = b * _CC + k
            for q in range(_Q):
                sl = pl.ds(q * _L, _L)
                gb[e, sl] = gb[e, sl] * rvv

    def _ss(b):
        pltpu.async_copy(gb.at[pl.ds(b * _CC, _CC)], agg_sh.at[rowsbuf.at[b]],
                         semOut[b], add=True)

    def _sw(b):
        pltpu.make_async_copy(gb.at[pl.ds(b * _CC, _CC)],
                              agg_sh.at[rowsbuf.at[b]], semOut[b]).wait()

    for b in range(_RING):
        _gs(b, b)

    @pl.loop(0, _NGRP)
    def _grp(p):
        for b in range(_RING):
            g = _RING * p + b
            _gw(g, b)
            _scale16(b)
            _ss(b)
        for b in range(_RING):
            g = _RING * p + b
            _sw(b)

            @pl.when(g + _RING < _NCHD)
            def _():
                _gs(g + _RING, b)

    glast = _NCHD - 1
    _gw(glast, 0)
    _scale16(0)
    _ss(0)
    _sw(0)

    plsc.subcore_barrier()

    @pl.loop(0, _NRCH)
    def _ocopy(k):
        r0 = ss * _ROWS_T + k * _RCH
        pltpu.sync_copy(agg_sh.at[pl.ds(r0, _RCH)], gb.at[pl.ds(0, _RCH)])
        pltpu.sync_copy(gb.at[pl.ds(0, _RCH)],
                        aggp_hbm.at[pl.ds(cc * _MP + r0, _RCH)])


@functools.partial(
    pl.kernel,
    out_type=(jax.ShapeDtypeStruct((_NW * _L,), jnp.float32),
              jax.ShapeDtypeStruct((_B,), jnp.float32)),
    mesh=_mesh(),
    scratch_types=[
        pltpu.VMEM((_C2,), jnp.int32),
        pltpu.VMEM((_C2,), jnp.int32),
        pltpu.VMEM((_C2,), jnp.int32),
        pltpu.VMEM((_C2,), jnp.int32),
        pltpu.VMEM((_C2,), jnp.int32),
        pltpu.VMEM((_C2,), jnp.int32),
        pltpu.VMEM((_C2, _D), jnp.float32),
        pltpu.VMEM((_C2, _D), jnp.float32),
        pltpu.VMEM((_C2, _D), jnp.float32),
        pltpu.VMEM((_C2, _D), jnp.float32),
        pltpu.VMEM((_C2, _D), jnp.float32),
        pltpu.VMEM((_C2, _D), jnp.float32),
        pltpu.VMEM((1, _D), jnp.float32),
        pltpu.VMEM((_PT,), jnp.float32),
        pltpu.VMEM((_L,), jnp.float32),
        pltpu.SemaphoreType.DMA,
        pltpu.SemaphoreType.DMA,
        pltpu.SemaphoreType.DMA,
        pltpu.SemaphoreType.DMA,
    ],
)
def _pairs(aggp_hbm, w_hbm, b_hbm, mu_hbm, i_hbm, j_hbm,
           pd_hbm, bj_hbm,
           iA, iB, i1A, i1B, jA, jB, a0A, a0B, a1A, a1B, wA, wB,
           mubuf, bjbuf, pdbuf, semIA, semIB, semGA, semGB):
    cc = lax.axis_index("c")
    ss = lax.axis_index("s")
    wid = cc * _NS + ss

    pltpu.sync_copy(mu_hbm, mubuf)

    def _idx_start(g, iX, jX, semIX):
        off = pl.multiple_of(wid * _PT + g * _C2, 8)
        pltpu.async_copy(i_hbm.at[pl.ds(off, _C2)], iX, semIX)
        pltpu.async_copy(j_hbm.at[pl.ds(off, _C2)], jX, semIX)

    def _idx_wait(g, iX, jX, semIX):
        off = pl.multiple_of(wid * _PT + g * _C2, 8)
        pltpu.make_async_copy(i_hbm.at[pl.ds(off, _C2)], iX, semIX).wait()
        pltpu.make_async_copy(j_hbm.at[pl.ds(off, _C2)], jX, semIX).wait()

    def _gath_start(g, iX, i1X, jX, a0X, a1X, wX, semGX):
        for t in range(_C2 // _L):
            sl = pl.ds(t * _L, _L)
            i1X[sl] = iX[sl] + _MP
        pltpu.async_copy(aggp_hbm.at[iX], a0X, semGX)
        pltpu.async_copy(aggp_hbm.at[i1X], a1X, semGX)
        pltpu.async_copy(w_hbm.at[jX], wX, semGX)
        pltpu.async_copy(b_hbm.at[jX],
                         bjbuf.at[pl.ds(g * _C2, _C2)], semGX)

    def _gath_wait(g, iX, i1X, jX, a0X, a1X, wX, semGX):
        pltpu.make_async_copy(aggp_hbm.at[iX], a0X, semGX).wait()
        pltpu.make_async_copy(aggp_hbm.at[i1X], a1X, semGX).wait()
        pltpu.make_async_copy(w_hbm.at[jX], wX, semGX).wait()
        pltpu.make_async_copy(b_hbm.at[jX],
                              bjbuf.at[pl.ds(g * _C2, _C2)], semGX).wait()

    def _compute(a0X, a1X, wX, acc):
        def _inner(e, acc):
            new = []
            for q in range(_Q):
                sl = pl.ds(q * _L, _L)
                x = a0X[e, sl] + a1X[e, sl] + mubuf[0, sl]
                h = 1.0 / (1.0 + jnp.exp(-x))
                new.append(acc[q] + h * wX[e, sl])
            return tuple(new)

        return lax.fori_loop(0, _C2, _inner, acc)

    _idx_start(0, iA, jA, semIA)
    _idx_wait(0, iA, jA, semIA)
    _gath_start(0, iA, i1A, jA, a0A, a1A, wA, semGA)
    _idx_start(1, iB, jB, semIB)
    _idx_wait(1, iB, jB, semIB)
    _gath_start(1, iB, i1B, jB, a0B, a1B, wB, semGB)

    acc0 = tuple(jnp.zeros((_L,), jnp.float32) for _ in range(_Q))

    @pl.loop(0, _NCH2 // 2, init_carry=acc0)
    def _pairloop(p, acc):
        g0 = 2 * p
        g1 = g0 + 1
        _gath_wait(g0, iA, i1A, jA, a0A, a1A, wA, semGA)

        @pl.when(g0 + 2 < _NCH2)
        def _():
            _idx_start(g0 + 2, iA, jA, semIA)

        acc = _compute(a0A, a1A, wA, acc)
        _gath_wait(g1, iB, i1B, jB, a0B, a1B, wB, semGB)

        @pl.when(g0 + 2 < _NCH2)
        def _():
            _idx_wait(g0 + 2, iA, jA, semIA)
            _gath_start(g0 + 2, iA, i1A, jA, a0A, a1A, wA, semGA)
            _idx_start(g1 + 2, iB, jB, semIB)

        acc = _compute(a0B, a1B, wB, acc)

        @pl.when(g1 + 2 < _NCH2)
        def _():
            _idx_wait(g1 + 2, iB, jB, semIB)
            _gath_start(g1 + 2, iB, i1B, jB, a0B, a1B, wB, semGB)

        return acc

    acc = _pairloop
    tot = acc[0]
    for q in range(1, _Q):
        tot = tot + acc[q]
    pdbuf[...] = tot
    pltpu.sync_copy(pdbuf, pd_hbm.at[pl.ds(wid * _L, _L)])
    pltpu.sync_copy(bjbuf, bj_hbm.at[pl.ds(wid * _PT, _PT)])


def kernel(ij, r, m, i, j, v, mu, w, b):
    del m
    ij = ij.astype(jnp.int32)
    rows = ij[0]
    cols = ij[1]
    aggp = _spmm(rows, cols, r.astype(jnp.float32), v)
    pd, bj = _pairs(aggp, w, b, mu, i.astype(jnp.int32), j.astype(jnp.int32))
    return jnp.sum(pd) + bj


# speedup vs baseline: 9.6351x; 1.0624x over previous
import functools

import jax
import jax.numpy as jnp
from jax import lax
from jax.experimental import pallas as pl
from jax.experimental.pallas import tpu as pltpu
from jax.experimental.pallas import tpu_sc as plsc

_N = 10000
_D = 128
_M = 10000
_NNZ = 320000
_B = 16384

_NC = 2
_NS = 16
_NW = _NC * _NS
_L = 16
_Q = _D // _L

_ET = _NNZ // _NW
_C1 = 80
_NCH1 = _ET // _C1
_NPAIR = (_NCH1 - 1) // 2
_ROWS_T = 640
_MP = _NS * _ROWS_T
_RCH = 128
_NRCH = _ROWS_T // _RCH

_PT = _B // _NW
_C2 = 64
_NCH2 = _PT // _C2


def _mesh():
    return plsc.VectorSubcoreMesh(
        core_axis_name="c", subcore_axis_name="s",
        num_cores=_NC, num_subcores=_NS)


_CC = 16
_NCHD = _ET // _CC
_RING = 8
_NGRP = 78


@functools.partial(
    pl.kernel,
    out_type=jax.ShapeDtypeStruct((_NC * _MP, _D), jnp.float32),
    mesh=_mesh(),
    scratch_types=[
        pltpu.VMEM_SHARED((_MP, _D), jnp.float32),
        pltpu.VMEM((_ET,), jnp.int32),
        pltpu.VMEM((_RING, _CC), jnp.int32),
        pltpu.VMEM((_RING, _CC), jnp.float32),
        pltpu.VMEM((_RING * _CC, _D), jnp.float32),
        pltpu.SemaphoreType.DMA,
        [pltpu.SemaphoreType.DMA] * _RING,
        [pltpu.SemaphoreType.DMA] * _RING,
    ],
)
def _spmm(ij_hbm, rv_hbm, v_hbm, aggp_hbm,
          agg_sh, cols_l, rowsbuf, rvbuf, gb, semi, semIn, semOut):
    cc = lax.axis_index("c")
    ss = lax.axis_index("s")
    wid = cc * _NS + ss
    ebase = wid * _ET

    pltpu.sync_copy(ij_hbm.at[pl.ds(_NNZ + ebase, _ET)], cols_l)

    def _gs(g, b):
        off = pl.multiple_of(g * _CC, 8)
        hoff = pl.multiple_of(ebase + g * _CC, 8)
        pltpu.async_copy(ij_hbm.at[pl.ds(hoff, _CC)], rowsbuf.at[b], semIn[b])
        pltpu.async_copy(rv_hbm.at[pl.ds(hoff, _CC)], rvbuf.at[b], semIn[b])
        pltpu.async_copy(v_hbm.at[cols_l.at[pl.ds(off, _CC)]],
                         gb.at[pl.ds(b * _CC, _CC)], semIn[b])

    def _gw(g, b):
        off = pl.multiple_of(g * _CC, 8)
        hoff = pl.multiple_of(ebase + g * _CC, 8)
        pltpu.make_async_copy(ij_hbm.at[pl.ds(hoff, _CC)], rowsbuf.at[b],
                              semIn[b]).wait()
        pltpu.make_async_copy(rv_hbm.at[pl.ds(hoff, _CC)], rvbuf.at[b],
                              semIn[b]).wait()
        pltpu.make_async_copy(v_hbm.at[cols_l.at[pl.ds(off, _CC)]],
                              gb.at[pl.ds(b * _CC, _CC)], semIn[b]).wait()

    def _scale16(b):
        rv16 = rvbuf[b, pl.ds(0, _CC)]
        for k in range(_CC):
            rvv = jnp.full((_L,), rv16[k], jnp.float32)
            e = b * _CC + k
            for q in range(_Q):
                sl = pl.ds(q * _L, _L)
                gb[e, sl] = gb[e, sl] * rvv

    def _ss(b):
        pltpu.async_copy(gb.at[pl.ds(b * _CC, _CC)], agg_sh.at[rowsbuf.at[b]],
                         semOut[b], add=True)

    def _sw(b):
        pltpu.make_async_copy(gb.at[pl.ds(b * _CC, _CC)],
                              agg_sh.at[rowsbuf.at[b]], semOut[b]).wait()

    for b in range(_RING // 2):
        _gs(b, b)

    @pl.loop(0, 64)
    def _zrow(rr):
        for q in range(_Q):
            gb[64 + rr, pl.ds(q * _L, _L)] = jnp.zeros((_L,), jnp.float32)

    zds = []
    for k in range(_ROWS_T // 64):
        r0 = ss * _ROWS_T + k * 64
        zds.append(pltpu.async_copy(gb.at[pl.ds(64, 64)],
                                    agg_sh.at[pl.ds(r0, 64)], semi))
    for d in zds:
        d.wait()
    plsc.subcore_barrier()

    for b in range(_RING // 2, _RING):
        _gs(b, b)

    @pl.loop(0, _NGRP)
    def _grp(p):
        for b in range(_RING):
            g = _RING * p + b
            _gw(g, b)
            _scale16(b)
            _ss(b)
        for b in range(_RING):
            g = _RING * p + b
            _sw(b)

            @pl.when(g + _RING < _NCHD)
            def _():
                _gs(g + _RING, b)

    glast = _NCHD - 1
    _gw(glast, 0)
    _scale16(0)
    _ss(0)
    _sw(0)

    plsc.subcore_barrier()

    NOC = _ROWS_T // 64
    din = [None] * NOC
    dout = [None] * NOC

    def _cin(k):
        r0 = ss * _ROWS_T + k * 64
        return pltpu.async_copy(agg_sh.at[pl.ds(r0, 64)],
                                gb.at[pl.ds((k % 2) * 64, 64)], semIn[k % 2])

    def _cout(k):
        r0 = ss * _ROWS_T + k * 64
        return pltpu.async_copy(gb.at[pl.ds((k % 2) * 64, 64)],
                                aggp_hbm.at[pl.ds(cc * _MP + r0, 64)],
                                semOut[k % 2])

    din[0] = _cin(0)
    for k in range(NOC):
        din[k].wait()
        if k >= 1:
            dout[k - 1].wait()
        if k + 1 < NOC:
            din[k + 1] = _cin(k + 1)
        dout[k] = _cout(k)
    dout[NOC - 1].wait()


@functools.partial(
    pl.kernel,
    out_type=(jax.ShapeDtypeStruct((_NW * _L,), jnp.float32),
              jax.ShapeDtypeStruct((_B,), jnp.float32)),
    mesh=_mesh(),
    scratch_types=[
        pltpu.VMEM((_C2,), jnp.int32),
        pltpu.VMEM((_C2,), jnp.int32),
        pltpu.VMEM((_C2,), jnp.int32),
        pltpu.VMEM((_C2,), jnp.int32),
        pltpu.VMEM((_C2,), jnp.int32),
        pltpu.VMEM((_C2,), jnp.int32),
        pltpu.VMEM((_C2, _D), jnp.float32),
        pltpu.VMEM((_C2, _D), jnp.float32),
        pltpu.VMEM((_C2, _D), jnp.float32),
        pltpu.VMEM((_C2, _D), jnp.float32),
        pltpu.VMEM((_C2, _D), jnp.float32),
        pltpu.VMEM((_C2, _D), jnp.float32),
        pltpu.VMEM((1, _D), jnp.float32),
        pltpu.VMEM((_PT,), jnp.float32),
        pltpu.VMEM((_L,), jnp.float32),
        pltpu.SemaphoreType.DMA,
        pltpu.SemaphoreType.DMA,
        pltpu.SemaphoreType.DMA,
        pltpu.SemaphoreType.DMA,
    ],
)
def _pairs(aggp_hbm, w_hbm, b_hbm, mu_hbm, i_hbm, j_hbm,
           pd_hbm, bj_hbm,
           iA, iB, i1A, i1B, jA, jB, a0A, a0B, a1A, a1B, wA, wB,
           mubuf, bjbuf, pdbuf, semIA, semIB, semGA, semGB):
    cc = lax.axis_index("c")
    ss = lax.axis_index("s")
    wid = cc * _NS + ss

    pltpu.sync_copy(mu_hbm, mubuf)

    def _idx_start(g, iX, jX, semIX):
        off = pl.multiple_of(wid * _PT + g * _C2, 8)
        pltpu.async_copy(i_hbm.at[pl.ds(off, _C2)], iX, semIX)
        pltpu.async_copy(j_hbm.at[pl.ds(off, _C2)], jX, semIX)

    def _idx_wait(g, iX, jX, semIX):
        off = pl.multiple_of(wid * _PT + g * _C2, 8)
        pltpu.make_async_copy(i_hbm.at[pl.ds(off, _C2)], iX, semIX).wait()
        pltpu.make_async_copy(j_hbm.at[pl.ds(off, _C2)], jX, semIX).wait()

    def _gath_start(g, iX, i1X, jX, a0X, a1X, wX, semGX):
        for t in range(_C2 // _L):
            sl = pl.ds(t * _L, _L)
            i1X[sl] = iX[sl] + _MP
        pltpu.async_copy(aggp_hbm.at[iX], a0X, semGX)
        pltpu.async_copy(aggp_hbm.at[i1X], a1X, semGX)
        pltpu.async_copy(w_hbm.at[jX], wX, semGX)
        pltpu.async_copy(b_hbm.at[jX],
                         bjbuf.at[pl.ds(g * _C2, _C2)], semGX)

    def _gath_wait(g, iX, i1X, jX, a0X, a1X, wX, semGX):
        pltpu.make_async_copy(aggp_hbm.at[iX], a0X, semGX).wait()
        pltpu.make_async_copy(aggp_hbm.at[i1X], a1X, semGX).wait()
        pltpu.make_async_copy(w_hbm.at[jX], wX, semGX).wait()
        pltpu.make_async_copy(b_hbm.at[jX],
                              bjbuf.at[pl.ds(g * _C2, _C2)], semGX).wait()

    def _compute(a0X, a1X, wX, acc):
        def _inner(e, acc):
            new = []
            for q in range(_Q):
                sl = pl.ds(q * _L, _L)
                x = a0X[e, sl] + a1X[e, sl] + mubuf[0, sl]
                h = 1.0 / (1.0 + jnp.exp(-x))
                new.append(acc[q] + h * wX[e, sl])
            return tuple(new)

        return lax.fori_loop(0, _C2, _inner, acc)

    _idx_start(0, iA, jA, semIA)
    _idx_wait(0, iA, jA, semIA)
    _gath_start(0, iA, i1A, jA, a0A, a1A, wA, semGA)
    _idx_start(1, iB, jB, semIB)
    _idx_wait(1, iB, jB, semIB)
    _gath_start(1, iB, i1B, jB, a0B, a1B, wB, semGB)

    acc0 = tuple(jnp.zeros((_L,), jnp.float32) for _ in range(_Q))

    @pl.loop(0, _NCH2 // 2, init_carry=acc0)
    def _pairloop(p, acc):
        g0 = 2 * p
        g1 = g0 + 1
        _gath_wait(g0, iA, i1A, jA, a0A, a1A, wA, semGA)

        @pl.when(g0 + 2 < _NCH2)
        def _():
            _idx_start(g0 + 2, iA, jA, semIA)

        acc = _compute(a0A, a1A, wA, acc)
        _gath_wait(g1, iB, i1B, jB, a0B, a1B, wB, semGB)

        @pl.when(g0 + 2 < _NCH2)
        def _():
            _idx_wait(g0 + 2, iA, jA, semIA)
            _gath_start(g0 + 2, iA, i1A, jA, a0A, a1A, wA, semGA)
            _idx_start(g1 + 2, iB, jB, semIB)

        acc = _compute(a0B, a1B, wB, acc)

        @pl.when(g1 + 2 < _NCH2)
        def _():
            _idx_wait(g1 + 2, iB, jB, semIB)
            _gath_start(g1 + 2, iB, i1B, jB, a0B, a1B, wB, semGB)

        return acc

    acc = _pairloop
    tot = acc[0]
    for q in range(1, _Q):
        tot = tot + acc[q]
    pdbuf[...] = tot
    pltpu.sync_copy(pdbuf, pd_hbm.at[pl.ds(wid * _L, _L)])
    pltpu.sync_copy(bjbuf, bj_hbm.at[pl.ds(wid * _PT, _PT)])


def kernel(ij, r, m, i, j, v, mu, w, b):
    del m
    ijf = ij.astype(jnp.int32).reshape(2 * _NNZ)
    aggp = _spmm(ijf, r.astype(jnp.float32), v)
    pd, bj = _pairs(aggp, w, b, mu, i.astype(jnp.int32), j.astype(jnp.int32))
    return jnp.sum(pd) + bj


# speedup vs baseline: 9.6483x; 1.0014x over previous
import functools

import jax
import jax.numpy as jnp
from jax import lax
from jax.experimental import pallas as pl
from jax.experimental.pallas import tpu as pltpu
from jax.experimental.pallas import tpu_sc as plsc

_N = 10000
_D = 128
_M = 10000
_NNZ = 320000
_B = 16384

_NC = 2
_NS = 16
_NW = _NC * _NS
_L = 16
_Q = _D // _L

_ET = _NNZ // _NW
_C1 = 80
_NCH1 = _ET // _C1
_NPAIR = (_NCH1 - 1) // 2
_ROWS_T = 640
_MP = _NS * _ROWS_T
_RCH = 128
_NRCH = _ROWS_T // _RCH

_PT = _B // _NW
_C2 = 64
_NCH2 = _PT // _C2


def _mesh():
    return plsc.VectorSubcoreMesh(
        core_axis_name="c", subcore_axis_name="s",
        num_cores=_NC, num_subcores=_NS)


_CC = 16
_NCHD = _ET // _CC
_RING = 8
_NGRP = 78


@functools.partial(
    pl.kernel,
    out_type=jax.ShapeDtypeStruct((_NC * _MP, _D), jnp.float32),
    mesh=_mesh(),
    scratch_types=[
        pltpu.VMEM_SHARED((_MP, _D), jnp.float32),
        pltpu.VMEM((_ET,), jnp.int32),
        pltpu.VMEM((_RING, _CC), jnp.int32),
        pltpu.VMEM((_RING, _CC), jnp.float32),
        pltpu.VMEM((_RING * _CC, _D), jnp.float32),
        pltpu.SemaphoreType.DMA,
        [pltpu.SemaphoreType.DMA] * _RING,
        [pltpu.SemaphoreType.DMA] * _RING,
    ],
    compiler_params=pltpu.CompilerParams(use_tc_tiling_on_sc=False),
)
def _spmm(ij_hbm, rv_hbm, v_hbm, aggp_hbm,
          agg_sh, cols_l, rowsbuf, rvbuf, gb, semi, semIn, semOut):
    cc = lax.axis_index("c")
    ss = lax.axis_index("s")
    wid = cc * _NS + ss
    ebase = wid * _ET

    pltpu.sync_copy(ij_hbm.at[1, pl.ds(ebase, _ET)], cols_l)

    def _gs(g, b):
        off = pl.multiple_of(g * _CC, 8)
        hoff = pl.multiple_of(ebase + g * _CC, 8)
        pltpu.async_copy(ij_hbm.at[0, pl.ds(hoff, _CC)], rowsbuf.at[b], semIn[b])
        pltpu.async_copy(rv_hbm.at[pl.ds(hoff, _CC)], rvbuf.at[b], semIn[b])
        pltpu.async_copy(v_hbm.at[cols_l.at[pl.ds(off, _CC)]],
                         gb.at[pl.ds(b * _CC, _CC)], semIn[b])

    def _gw(g, b):
        off = pl.multiple_of(g * _CC, 8)
        hoff = pl.multiple_of(ebase + g * _CC, 8)
        pltpu.make_async_copy(ij_hbm.at[0, pl.ds(hoff, _CC)], rowsbuf.at[b],
                              semIn[b]).wait()
        pltpu.make_async_copy(rv_hbm.at[pl.ds(hoff, _CC)], rvbuf.at[b],
                              semIn[b]).wait()
        pltpu.make_async_copy(v_hbm.at[cols_l.at[pl.ds(off, _CC)]],
                              gb.at[pl.ds(b * _CC, _CC)], semIn[b]).wait()

    def _scale16(b):
        rv16 = rvbuf[b, pl.ds(0, _CC)]
        for k in range(_CC):
            rvv = jnp.full((_L,), rv16[k], jnp.float32)
            e = b * _CC + k
            for q in range(_Q):
                sl = pl.ds(q * _L, _L)
                gb[e, sl] = gb[e, sl] * rvv

    def _ss(b):
        pltpu.async_copy(gb.at[pl.ds(b * _CC, _CC)], agg_sh.at[rowsbuf.at[b]],
                         semOut[b], add=True)

    def _sw(b):
        pltpu.make_async_copy(gb.at[pl.ds(b * _CC, _CC)],
                              agg_sh.at[rowsbuf.at[b]], semOut[b]).wait()

    for b in range(_RING // 2):
        _gs(b, b)

    @pl.loop(0, 64)
    def _zrow(rr):
        for q in range(_Q):
            gb[64 + rr, pl.ds(q * _L, _L)] = jnp.zeros((_L,), jnp.float32)

    zds = []
    for k in range(_ROWS_T // 64):
        r0 = ss * _ROWS_T + k * 64
        zds.append(pltpu.async_copy(gb.at[pl.ds(64, 64)],
                                    agg_sh.at[pl.ds(r0, 64)], semi))
    for d in zds:
        d.wait()
    plsc.subcore_barrier()

    for b in range(_RING // 2, _RING):
        _gs(b, b)

    @pl.loop(0, _NGRP)
    def _grp(p):
        for b in range(_RING):
            g = _RING * p + b
            _gw(g, b)
            _scale16(b)
            _ss(b)
        for b in range(_RING):
            g = _RING * p + b
            _sw(b)

            @pl.when(g + _RING < _NCHD)
            def _():
                _gs(g + _RING, b)

    glast = _NCHD - 1
    _gw(glast, 0)
    _scale16(0)
    _ss(0)
    _sw(0)

    plsc.subcore_barrier()

    NOC = _ROWS_T // 64
    din = [None] * NOC
    dout = [None] * NOC

    def _cin(k):
        r0 = ss * _ROWS_T + k * 64
        return pltpu.async_copy(agg_sh.at[pl.ds(r0, 64)],
                                gb.at[pl.ds((k % 2) * 64, 64)], semIn[k % 2])

    def _cout(k):
        r0 = ss * _ROWS_T + k * 64
        return pltpu.async_copy(gb.at[pl.ds((k % 2) * 64, 64)],
                                aggp_hbm.at[pl.ds(cc * _MP + r0, 64)],
                                semOut[k % 2])

    din[0] = _cin(0)
    for k in range(NOC):
        din[k].wait()
        if k >= 1:
            dout[k - 1].wait()
        if k + 1 < NOC:
            din[k + 1] = _cin(k + 1)
        dout[k] = _cout(k)
    dout[NOC - 1].wait()


@functools.partial(
    pl.kernel,
    out_type=(jax.ShapeDtypeStruct((_NW * _L,), jnp.float32),
              jax.ShapeDtypeStruct((_B,), jnp.float32)),
    mesh=_mesh(),
    scratch_types=[
        pltpu.VMEM((_C2,), jnp.int32),
        pltpu.VMEM((_C2,), jnp.int32),
        pltpu.VMEM((_C2,), jnp.int32),
        pltpu.VMEM((_C2,), jnp.int32),
        pltpu.VMEM((_C2,), jnp.int32),
        pltpu.VMEM((_C2,), jnp.int32),
        pltpu.VMEM((_C2, _D), jnp.float32),
        pltpu.VMEM((_C2, _D), jnp.float32),
        pltpu.VMEM((_C2, _D), jnp.float32),
        pltpu.VMEM((_C2, _D), jnp.float32),
        pltpu.VMEM((_C2, _D), jnp.float32),
        pltpu.VMEM((_C2, _D), jnp.float32),
        pltpu.VMEM((1, _D), jnp.float32),
        pltpu.VMEM((_PT,), jnp.float32),
        pltpu.VMEM((_L,), jnp.float32),
        pltpu.SemaphoreType.DMA,
        pltpu.SemaphoreType.DMA,
        pltpu.SemaphoreType.DMA,
        pltpu.SemaphoreType.DMA,
    ],
)
def _pairs(aggp_hbm, w_hbm, b_hbm, mu_hbm, i_hbm, j_hbm,
           pd_hbm, bj_hbm,
           iA, iB, i1A, i1B, jA, jB, a0A, a0B, a1A, a1B, wA, wB,
           mubuf, bjbuf, pdbuf, semIA, semIB, semGA, semGB):
    cc = lax.axis_index("c")
    ss = lax.axis_index("s")
    wid = cc * _NS + ss

    pltpu.sync_copy(mu_hbm, mubuf)

    def _idx_start(g, iX, jX, semIX):
        off = pl.multiple_of(wid * _PT + g * _C2, 8)
        pltpu.async_copy(i_hbm.at[pl.ds(off, _C2)], iX, semIX)
        pltpu.async_copy(j_hbm.at[pl.ds(off, _C2)], jX, semIX)

    def _idx_wait(g, iX, jX, semIX):
        off = pl.multiple_of(wid * _PT + g * _C2, 8)
        pltpu.make_async_copy(i_hbm.at[pl.ds(off, _C2)], iX, semIX).wait()
        pltpu.make_async_copy(j_hbm.at[pl.ds(off, _C2)], jX, semIX).wait()

    def _gath_start(g, iX, i1X, jX, a0X, a1X, wX, semGX):
        for t in range(_C2 // _L):
            sl = pl.ds(t * _L, _L)
            i1X[sl] = iX[sl] + _MP
        pltpu.async_copy(aggp_hbm.at[iX], a0X, semGX)
        pltpu.async_copy(aggp_hbm.at[i1X], a1X, semGX)
        pltpu.async_copy(w_hbm.at[jX], wX, semGX)
        pltpu.async_copy(b_hbm.at[jX],
                         bjbuf.at[pl.ds(g * _C2, _C2)], semGX)

    def _gath_wait(g, iX, i1X, jX, a0X, a1X, wX, semGX):
        pltpu.make_async_copy(aggp_hbm.at[iX], a0X, semGX).wait()
        pltpu.make_async_copy(aggp_hbm.at[i1X], a1X, semGX).wait()
        pltpu.make_async_copy(w_hbm.at[jX], wX, semGX).wait()
        pltpu.make_async_copy(b_hbm.at[jX],
                              bjbuf.at[pl.ds(g * _C2, _C2)], semGX).wait()

    def _compute(a0X, a1X, wX, acc):
        def _inner(e, acc):
            new = []
            for q in range(_Q):
                sl = pl.ds(q * _L, _L)
                x = a0X[e, sl] + a1X[e, sl] + mubuf[0, sl]
                h = 1.0 / (1.0 + jnp.exp(-x))
                new.append(acc[q] + h * wX[e, sl])
            return tuple(new)

        return lax.fori_loop(0, _C2, _inner, acc)

    _idx_start(0, iA, jA, semIA)
    _idx_wait(0, iA, jA, semIA)
    _gath_start(0, iA, i1A, jA, a0A, a1A, wA, semGA)
    _idx_start(1, iB, jB, semIB)
    _idx_wait(1, iB, jB, semIB)
    _gath_start(1, iB, i1B, jB, a0B, a1B, wB, semGB)

    acc0 = tuple(jnp.zeros((_L,), jnp.float32) for _ in range(_Q))

    @pl.loop(0, _NCH2 // 2, init_carry=acc0)
    def _pairloop(p, acc):
        g0 = 2 * p
        g1 = g0 + 1
        _gath_wait(g0, iA, i1A, jA, a0A, a1A, wA, semGA)

        @pl.when(g0 + 2 < _NCH2)
        def _():
            _idx_start(g0 + 2, iA, jA, semIA)

        acc = _compute(a0A, a1A, wA, acc)
        _gath_wait(g1, iB, i1B, jB, a0B, a1B, wB, semGB)

        @pl.when(g0 + 2 < _NCH2)
        def _():
            _idx_wait(g0 + 2, iA, jA, semIA)
            _gath_start(g0 + 2, iA, i1A, jA, a0A, a1A, wA, semGA)
            _idx_start(g1 + 2, iB, jB, semIB)

        acc = _compute(a0B, a1B, wB, acc)

        @pl.when(g1 + 2 < _NCH2)
        def _():
            _idx_wait(g1 + 2, iB, jB, semIB)
            _gath_start(g1 + 2, iB, i1B, jB, a0B, a1B, wB, semGB)

        return acc

    acc = _pairloop
    tot = acc[0]
    for q in range(1, _Q):
        tot = tot + acc[q]
    pdbuf[...] = tot
    pltpu.sync_copy(pdbuf, pd_hbm.at[pl.ds(wid * _L, _L)])
    pltpu.sync_copy(bjbuf, bj_hbm.at[pl.ds(wid * _PT, _PT)])


def kernel(ij, r, m, i, j, v, mu, w, b):
    del m
    aggp = _spmm(ij.astype(jnp.int32), r.astype(jnp.float32), v)
    pd, bj = _pairs(aggp, w, b, mu, i.astype(jnp.int32), j.astype(jnp.int32))
    return jnp.sum(pd) + bj
